# baseline (speedup 1.0000x reference)
import functools
import math

import jax
import jax.numpy as jnp
from jax.experimental import pallas as pl
from jax.experimental.pallas import tpu as pltpu


def _level_body(H, L, Bp, has_child, *refs):
    if has_child:
        (x, ss, sc, dh_in, dc_in, xp,
         W_iou, U_iou, b_iou, W_f, U_f, b_f, hacc_in,
         dh_out, dc_out, hacc_out) = refs
    else:
        (x, ss, sc, xp,
         W_iou, U_iou, b_iou, W_f, U_f, b_f, hacc_in,
         dh_out, dc_out, hacc_out) = refs
        dh_in = dc_in = None

    i = pl.program_id(0)
    B2 = 2 * Bp
    rows = jax.lax.broadcasted_iota(jnp.int32, (B2, 1), 0) + i * B2
    m = jnp.where(rows < L, 1.0, 0.0).astype(jnp.float32)

    hsum = ss[:, :]
    cin = sc[:, :]
    if dh_in is not None:
        hsum = hsum + dh_in[:, :]
        cin = cin + dc_in[:, :]
    iou = (jnp.dot(x[:, :], W_iou[:, :], preferred_element_type=jnp.float32)
           + b_iou[:, :]
           + jnp.dot(hsum, U_iou[:, :], preferred_element_type=jnp.float32))
    i_g = iou[:, :H]
    o_g = iou[:, H:2 * H]
    u_g = iou[:, 2 * H:]
    c = jax.nn.sigmoid(i_g) * jnp.tanh(u_g) + cin
    h = jax.nn.sigmoid(o_g) * jnp.tanh(c)
    h = h * m
    c = c * m

    xpW = jnp.dot(xp[:, :], W_f[:, :], preferred_element_type=jnp.float32) + b_f[:, :]
    xpW2 = jnp.broadcast_to(xpW[:, None, :], (Bp, 2, H)).reshape(B2, H)
    f = jax.nn.sigmoid(xpW2 + jnp.dot(h, U_f[:, :],
                                      preferred_element_type=jnp.float32))
    dh_out[:, :] = h.reshape(Bp, 2, H).sum(axis=1)
    dc_out[:, :] = (f * c).reshape(Bp, 2, H).sum(axis=1)
    part = jnp.sum(h, axis=0, keepdims=True)

    @pl.when(i == 0)
    def _():
        hacc_out[:, :] = hacc_in[:, :] + part

    @pl.when(i > 0)
    def _():
        hacc_out[:, :] = hacc_out[:, :] + part


def _root_body(H, x, ss, sc, dh, dc, W_iou, U_iou, b_iou, hacc_in,
               mu_out, lv_out):
    hsum = ss[:, :] + dh[:, :]
    cin = sc[:, :] + dc[:, :]
    iou = (jnp.dot(x[:, :], W_iou[:, :], preferred_element_type=jnp.float32)
           + b_iou[:, :]
           + jnp.dot(hsum, U_iou[:, :], preferred_element_type=jnp.float32))
    i_g = iou[:, :H]
    o_g = iou[:, H:2 * H]
    u_g = iou[:, 2 * H:]
    c = jax.nn.sigmoid(i_g) * jnp.tanh(u_g) + cin
    h = jax.nn.sigmoid(o_g) * jnp.tanh(c)
    htot = hacc_in[:, :] + h[0:1, :]
    mu_out[:, :] = htot[:, :H // 2]
    lv_out[:, :] = jnp.tanh(htot[:, H // 2:])


def _pad_rows(a, rows):
    if a.shape[0] == rows:
        return a
    return jnp.pad(a, ((0, rows - a.shape[0]), (0, 0)))


def kernel(embed, edge_index, structure_sum, structure_c,
           W_iou, U_iou, b_iou, W_f, U_f, b_f):
    del edge_index
    n = embed.shape[0]
    in_dim = embed.shape[1]
    H = U_f.shape[0]
    f32 = jnp.float32

    b_iou2 = b_iou.reshape(1, 3 * H)
    b_f2 = b_f.reshape(1, H)

    max_d = int(math.floor(math.log2(n)))
    hacc = jnp.zeros((1, H), f32)
    dh = None
    dc = None

    for d in range(max_d, 0, -1):
        s = 2 ** d - 1
        e = min(2 ** (d + 1) - 1, n)
        L = e - s
        Lp = (L + 1) // 2
        Bp = min(1024, max(4, -(-Lp // 4) * 4))
        B2 = 2 * Bp
        G = -(-Lp // Bp)
        P = G * Bp

        x = _pad_rows(embed[s:e], G * B2)
        ss = _pad_rows(structure_sum[s:e], G * B2)
        sc = _pad_rows(structure_c[s:e], G * B2)
        sp = 2 ** (d - 1) - 1
        xp = _pad_rows(embed[sp:sp + Lp], P)

        has_child = dh is not None
        if has_child:
            dhl = _pad_rows(dh, G * B2)
            dcl = _pad_rows(dc, G * B2)
            data_args = (x, ss, sc, dhl, dcl, xp)
        else:
            data_args = (x, ss, sc, xp)

        def ctile(cols):
            return pl.BlockSpec((B2, cols), lambda i: (i, 0))

        def ptile(cols):
            return pl.BlockSpec((Bp, cols), lambda i: (i, 0))

        def full(shape):
            return pl.BlockSpec(shape, lambda i: (0,) * len(shape))

        n_data = len(data_args)
        in_specs = (
            [ctile(in_dim)]
            + [ctile(H)] * (n_data - 2)
            + [ptile(in_dim)]
            + [full((in_dim, 3 * H)), full((H, 3 * H)), full((1, 3 * H)),
               full((in_dim, H)), full((H, H)), full((1, H)),
               full((1, H))]
        )
        out_specs = [ptile(H), ptile(H), full((1, H))]
        out_shape = [jax.ShapeDtypeStruct((P, H), f32),
                     jax.ShapeDtypeStruct((P, H), f32),
                     jax.ShapeDtypeStruct((1, H), f32)]

        body = functools.partial(_level_body, H, L, Bp, has_child)
        dh_full, dc_full, hacc = pl.pallas_call(
            body,
            grid=(G,),
            in_specs=in_specs,
            out_specs=out_specs,
            out_shape=out_shape,
            compiler_params=pltpu.CompilerParams(
                dimension_semantics=("arbitrary",)),
        )(*data_args, W_iou, U_iou, b_iou2, W_f, U_f, b_f2, hacc)

        dh = dh_full[:Lp]
        dc = dc_full[:Lp]

    x_r = _pad_rows(embed[0:1], 8)
    ss_r = _pad_rows(structure_sum[0:1], 8)
    sc_r = _pad_rows(structure_c[0:1], 8)
    dh_r = _pad_rows(dh, 8)
    dc_r = _pad_rows(dc, 8)
    mu, lv = pl.pallas_call(
        functools.partial(_root_body, H),
        out_shape=[jax.ShapeDtypeStruct((1, H // 2), f32),
                   jax.ShapeDtypeStruct((1, H // 2), f32)],
    )(x_r, ss_r, sc_r, dh_r, dc_r, W_iou, U_iou, b_iou2, hacc)
    return (mu, lv)


# speedup vs baseline: 1.0028x; 1.0028x over previous
import functools
import math

import jax
import jax.numpy as jnp
from jax.experimental import pallas as pl
from jax.experimental.pallas import tpu as pltpu


def _level_body(H, L, Bp, has_child, *refs):
    if has_child:
        (x, ss, sc, dh_in, dc_in, xp,
         W_iou, U_iou, b_iou, W_f, U_f, b_f, hacc_in,
         dh_out, dc_out, hacc_out) = refs
    else:
        (x, ss, sc, xp,
         W_iou, U_iou, b_iou, W_f, U_f, b_f, hacc_in,
         dh_out, dc_out, hacc_out) = refs
        dh_in = dc_in = None

    i = pl.program_id(0)
    B2 = 2 * Bp
    rows = jax.lax.broadcasted_iota(jnp.int32, (B2, 1), 0) + i * B2
    m = jnp.where(rows < L, 1.0, 0.0).astype(jnp.float32)

    hsum = ss[:, :]
    cin = sc[:, :]
    if dh_in is not None:
        hsum = hsum + dh_in[:, :]
        cin = cin + dc_in[:, :]
    iou = (jnp.dot(x[:, :], W_iou[:, :], preferred_element_type=jnp.float32)
           + b_iou[:, :]
           + jnp.dot(hsum, U_iou[:, :], preferred_element_type=jnp.float32))
    i_g = iou[:, :H]
    o_g = iou[:, H:2 * H]
    u_g = iou[:, 2 * H:]
    c = jax.nn.sigmoid(i_g) * jnp.tanh(u_g) + cin
    h = jax.nn.sigmoid(o_g) * jnp.tanh(c)
    h = h * m
    c = c * m

    xpW = jnp.dot(xp[:, :], W_f[:, :], preferred_element_type=jnp.float32) + b_f[:, :]
    xpW2 = jnp.broadcast_to(xpW[:, None, :], (Bp, 2, H)).reshape(B2, H)
    f = jax.nn.sigmoid(xpW2 + jnp.dot(h, U_f[:, :],
                                      preferred_element_type=jnp.float32))
    dh_out[:, :] = h.reshape(Bp, 2, H).sum(axis=1)
    dc_out[:, :] = (f * c).reshape(Bp, 2, H).sum(axis=1)
    part = jnp.sum(h, axis=0, keepdims=True)

    @pl.when(i == 0)
    def _():
        hacc_out[:, :] = hacc_in[:, :] + part

    @pl.when(i > 0)
    def _():
        hacc_out[:, :] = hacc_out[:, :] + part


def _root_body(H, x, ss, sc, dh, dc, W_iou, U_iou, b_iou, hacc_in,
               mu_out, lv_out):
    hsum = ss[:, :] + dh[:, :]
    cin = sc[:, :] + dc[:, :]
    iou = (jnp.dot(x[:, :], W_iou[:, :], preferred_element_type=jnp.float32)
           + b_iou[:, :]
           + jnp.dot(hsum, U_iou[:, :], preferred_element_type=jnp.float32))
    i_g = iou[:, :H]
    o_g = iou[:, H:2 * H]
    u_g = iou[:, 2 * H:]
    c = jax.nn.sigmoid(i_g) * jnp.tanh(u_g) + cin
    h = jax.nn.sigmoid(o_g) * jnp.tanh(c)
    htot = hacc_in[:, :] + h[0:1, :]
    mu_out[:, :] = htot[:, :H // 2]
    lv_out[:, :] = jnp.tanh(htot[:, H // 2:])


def _pad_rows(a, rows):
    if a.shape[0] == rows:
        return a
    return jnp.pad(a, ((0, rows - a.shape[0]), (0, 0)))


def kernel(embed, edge_index, structure_sum, structure_c,
           W_iou, U_iou, b_iou, W_f, U_f, b_f):
    del edge_index
    n = embed.shape[0]
    in_dim = embed.shape[1]
    H = U_f.shape[0]
    f32 = jnp.float32

    b_iou2 = b_iou.reshape(1, 3 * H)
    b_f2 = b_f.reshape(1, H)

    max_d = int(math.floor(math.log2(n)))
    hacc = jnp.zeros((1, H), f32)
    dh = None
    dc = None

    for d in range(max_d, 0, -1):
        s = 2 ** d - 1
        e = min(2 ** (d + 1) - 1, n)
        L = e - s
        Lp = (L + 1) // 2
        Bp = min(512, max(4, -(-Lp // 4) * 4))
        B2 = 2 * Bp
        G = -(-Lp // Bp)
        P = G * Bp

        x = _pad_rows(embed[s:e], G * B2)
        ss = _pad_rows(structure_sum[s:e], G * B2)
        sc = _pad_rows(structure_c[s:e], G * B2)
        sp = 2 ** (d - 1) - 1
        xp = _pad_rows(embed[sp:sp + Lp], P)

        has_child = dh is not None
        if has_child:
            dhl = _pad_rows(dh, G * B2)
            dcl = _pad_rows(dc, G * B2)
            data_args = (x, ss, sc, dhl, dcl, xp)
        else:
            data_args = (x, ss, sc, xp)

        def ctile(cols):
            return pl.BlockSpec((B2, cols), lambda i: (i, 0))

        def ptile(cols):
            return pl.BlockSpec((Bp, cols), lambda i: (i, 0))

        def full(shape):
            return pl.BlockSpec(shape, lambda i: (0,) * len(shape))

        n_data = len(data_args)
        in_specs = (
            [ctile(in_dim)]
            + [ctile(H)] * (n_data - 2)
            + [ptile(in_dim)]
            + [full((in_dim, 3 * H)), full((H, 3 * H)), full((1, 3 * H)),
               full((in_dim, H)), full((H, H)), full((1, H)),
               full((1, H))]
        )
        out_specs = [ptile(H), ptile(H), full((1, H))]
        out_shape = [jax.ShapeDtypeStruct((P, H), f32),
                     jax.ShapeDtypeStruct((P, H), f32),
                     jax.ShapeDtypeStruct((1, H), f32)]

        body = functools.partial(_level_body, H, L, Bp, has_child)
        dh_full, dc_full, hacc = pl.pallas_call(
            body,
            grid=(G,),
            in_specs=in_specs,
            out_specs=out_specs,
            out_shape=out_shape,
            compiler_params=pltpu.CompilerParams(
                dimension_semantics=("arbitrary",)),
        )(*data_args, W_iou, U_iou, b_iou2, W_f, U_f, b_f2, hacc)

        dh = dh_full[:Lp]
        dc = dc_full[:Lp]

    x_r = _pad_rows(embed[0:1], 8)
    ss_r = _pad_rows(structure_sum[0:1], 8)
    sc_r = _pad_rows(structure_c[0:1], 8)
    dh_r = _pad_rows(dh, 8)
    dc_r = _pad_rows(dc, 8)
    mu, lv = pl.pallas_call(
        functools.partial(_root_body, H),
        out_shape=[jax.ShapeDtypeStruct((1, H // 2), f32),
                   jax.ShapeDtypeStruct((1, H // 2), f32)],
    )(x_r, ss_r, sc_r, dh_r, dc_r, W_iou, U_iou, b_iou2, hacc)
    return (mu, lv)


# speedup vs baseline: 1.5681x; 1.5638x over previous
import functools
import math

import jax
import jax.numpy as jnp
from jax.experimental import pallas as pl
from jax.experimental.pallas import tpu as pltpu

_SH = 7


def _level_body(H, in_dim, L, C, Bp, G, s, sp, tail, only_ops, has_child,
                *refs):
    B2 = 2 * Bp
    it = iter(refs)
    embed = next(it)
    ss_h = next(it)
    sc_h = next(it)
    if tail:
        xt, sst, sct, xpt = next(it), next(it), next(it), next(it)
    if has_child:
        dh_in, dc_in = next(it), next(it)
    W_iou, U_iou, b_iou, W_f, U_f, b_f, hacc_in = (
        next(it), next(it), next(it), next(it), next(it), next(it), next(it))
    dh_out, dc_out, hacc_out = next(it), next(it), next(it)
    x_buf, ss_buf, sc_buf, xp_buf, sems = (
        next(it), next(it), next(it), next(it), next(it))

    def copy_child(j, slot):
        def from_main():
            base = s - _SH + j * B2
            pltpu.make_async_copy(embed.at[pl.ds(base, B2 + 8), :],
                                  x_buf.at[slot], sems.at[slot, 0]).start()
            pltpu.make_async_copy(ss_h.at[pl.ds(base, B2 + 8), :],
                                  ss_buf.at[slot], sems.at[slot, 1]).start()
            pltpu.make_async_copy(sc_h.at[pl.ds(base, B2 + 8), :],
                                  sc_buf.at[slot], sems.at[slot, 2]).start()
            pltpu.make_async_copy(embed.at[pl.ds(sp - _SH + j * Bp, Bp + 8), :],
                                  xp_buf.at[slot], sems.at[slot, 3]).start()

        def from_ops():
            pltpu.make_async_copy(xt.at[:, :], x_buf.at[slot],
                                  sems.at[slot, 0]).start()
            pltpu.make_async_copy(sst.at[:, :], ss_buf.at[slot],
                                  sems.at[slot, 1]).start()
            pltpu.make_async_copy(sct.at[:, :], sc_buf.at[slot],
                                  sems.at[slot, 2]).start()
            pltpu.make_async_copy(xpt.at[:, :], xp_buf.at[slot],
                                  sems.at[slot, 3]).start()

        if only_ops:
            from_ops()
        elif tail:
            @pl.when(j < G - 1)
            def _():
                from_main()

            @pl.when(j == G - 1)
            def _():
                from_ops()
        else:
            from_main()

    def wait_child(slot):
        pltpu.make_async_copy(embed.at[pl.ds(0, B2 + 8), :],
                              x_buf.at[slot], sems.at[slot, 0]).wait()
        pltpu.make_async_copy(ss_h.at[pl.ds(0, B2 + 8), :],
                              ss_buf.at[slot], sems.at[slot, 1]).wait()
        pltpu.make_async_copy(sc_h.at[pl.ds(0, B2 + 8), :],
                              sc_buf.at[slot], sems.at[slot, 2]).wait()
        pltpu.make_async_copy(embed.at[pl.ds(0, Bp + 8), :],
                              xp_buf.at[slot], sems.at[slot, 3]).wait()

    i = pl.program_id(0)
    slot = jax.lax.rem(i, 2)

    @pl.when(i == 0)
    def _():
        copy_child(0, 0)

    @pl.when(i + 1 < G)
    def _():
        copy_child(i + 1, jax.lax.rem(i + 1, 2))

    wait_child(slot)

    rows = jax.lax.broadcasted_iota(jnp.int32, (B2, 1), 0) + i * B2
    m = rows < L

    hsum = ss_buf[slot][_SH:_SH + B2]
    cin = sc_buf[slot][_SH:_SH + B2]
    if has_child:
        m_c = rows < C
        hsum = hsum + jnp.where(m_c, dh_in[:, :], 0.0)
        cin = cin + jnp.where(m_c, dc_in[:, :], 0.0)
    x = x_buf[slot][_SH:_SH + B2]
    iou = (jnp.dot(x, W_iou[:, :], preferred_element_type=jnp.float32)
           + b_iou[:, :]
           + jnp.dot(hsum, U_iou[:, :], preferred_element_type=jnp.float32))
    i_g = iou[:, :H]
    o_g = iou[:, H:2 * H]
    u_g = iou[:, 2 * H:]
    c = jax.nn.sigmoid(i_g) * jnp.tanh(u_g) + cin
    h = jax.nn.sigmoid(o_g) * jnp.tanh(c)
    h = jnp.where(m, h, 0.0)

    xpW = jnp.dot(xp_buf[slot][_SH:_SH + Bp], W_f[:, :],
                  preferred_element_type=jnp.float32) + b_f[:, :]
    xpW2 = jnp.broadcast_to(xpW[:, None, :], (Bp, 2, H)).reshape(B2, H)
    f = jax.nn.sigmoid(xpW2 + jnp.dot(h, U_f[:, :],
                                      preferred_element_type=jnp.float32))
    fc = jnp.where(m, f * c, 0.0)
    dh_out[:, :] = h.reshape(Bp, 2, H).sum(axis=1)
    dc_out[:, :] = fc.reshape(Bp, 2, H).sum(axis=1)
    part = jnp.sum(h, axis=0, keepdims=True)

    @pl.when(i == 0)
    def _():
        hacc_out[:, :] = hacc_in[:, :] + part

    @pl.when(i > 0)
    def _():
        hacc_out[:, :] = hacc_out[:, :] + part


def _root_body(H, x, ss, sc, dh, dc, W_iou, U_iou, b_iou, hacc_in,
               mu_out, lv_out):
    hsum = ss[0:1, :] + dh[0:1, :]
    cin = sc[0:1, :] + dc[0:1, :]
    iou = (jnp.dot(x[0:1, :], W_iou[:, :], preferred_element_type=jnp.float32)
           + b_iou[:, :]
           + jnp.dot(hsum, U_iou[:, :], preferred_element_type=jnp.float32))
    i_g = iou[:, :H]
    o_g = iou[:, H:2 * H]
    u_g = iou[:, 2 * H:]
    c = jax.nn.sigmoid(i_g) * jnp.tanh(u_g) + cin
    h = jax.nn.sigmoid(o_g) * jnp.tanh(c)
    htot = hacc_in[:, :] + h
    mu_out[:, :] = htot[:, :H // 2]
    lv_out[:, :] = jnp.tanh(htot[:, H // 2:])


def _pad_rows(a, rows):
    if a.shape[0] == rows:
        return a
    return jnp.pad(a, ((0, rows - a.shape[0]), (0, 0)))


def _round_up(v, m):
    return -(-v // m) * m


def kernel(embed, edge_index, structure_sum, structure_c,
           W_iou, U_iou, b_iou, W_f, U_f, b_f):
    del edge_index
    n = embed.shape[0]
    in_dim = embed.shape[1]
    H = U_f.shape[0]
    f32 = jnp.float32

    b_iou2 = b_iou.reshape(1, 3 * H)
    b_f2 = b_f.reshape(1, H)

    def padded_slice(arr, start, rows):
        lead = max(0, -start)
        s0 = max(0, start)
        e0 = min(n, start + rows)
        return jnp.pad(arr[s0:e0], ((lead, rows - (e0 - s0) - lead), (0, 0)))

    max_d = int(math.floor(math.log2(n)))

    levels = []
    for d in range(max_d, 0, -1):
        s = 2 ** d - 1
        e = min(2 ** (d + 1) - 1, n)
        L = e - s
        Lp = (L + 1) // 2
        Bp = min(512, max(8, _round_up(Lp, 8)))
        B2 = 2 * Bp
        G = -(-Lp // Bp)
        levels.append(dict(d=d, s=s, e=e, L=L, Lp=Lp, Bp=Bp, B2=B2, G=G,
                           sp=2 ** (d - 1) - 1))
    for k, lv in enumerate(levels):
        need = levels[k + 1]["G"] * levels[k + 1]["B2"] if k + 1 < len(levels) else 8
        lv["P_out"] = max(need, lv["G"] * lv["Bp"])

    hacc = jnp.zeros((1, H), f32)
    dh = None
    dc = None
    C = 0

    hbm = pl.BlockSpec(memory_space=pltpu.MemorySpace.HBM)

    for lv in levels:
        s, L, Lp, Bp, B2, G, sp, P_out = (lv["s"], lv["L"], lv["Lp"], lv["Bp"],
                                          lv["B2"], lv["G"], lv["sp"], lv["P_out"])
        tail = (s - _SH + G * B2 + 8 > n) or (s - _SH < 0) or (sp - _SH < 0)
        only_ops = tail and G == 1
        has_child = dh is not None

        args = [embed, structure_sum, structure_c]
        in_specs = [hbm, hbm, hbm]
        if tail:
            tb = s - _SH + (G - 1) * B2
            args += [padded_slice(embed, tb, B2 + 8),
                     padded_slice(structure_sum, tb, B2 + 8),
                     padded_slice(structure_c, tb, B2 + 8),
                     padded_slice(embed, sp - _SH + (G - 1) * Bp, Bp + 8)]
            in_specs += [hbm, hbm, hbm, hbm]
        if has_child:
            args += [dh, dc]
            in_specs += [pl.BlockSpec((B2, H), lambda i: (i, 0)),
                         pl.BlockSpec((B2, H), lambda i: (i, 0))]
        args += [W_iou, U_iou, b_iou2, W_f, U_f, b_f2, hacc]
        in_specs += [pl.BlockSpec((in_dim, 3 * H), lambda i: (0, 0)),
                     pl.BlockSpec((H, 3 * H), lambda i: (0, 0)),
                     pl.BlockSpec((1, 3 * H), lambda i: (0, 0)),
                     pl.BlockSpec((in_dim, H), lambda i: (0, 0)),
                     pl.BlockSpec((H, H), lambda i: (0, 0)),
                     pl.BlockSpec((1, H), lambda i: (0, 0)),
                     pl.BlockSpec((1, H), lambda i: (0, 0))]

        out_specs = [pl.BlockSpec((Bp, H), lambda i: (i, 0)),
                     pl.BlockSpec((Bp, H), lambda i: (i, 0)),
                     pl.BlockSpec((1, H), lambda i: (0, 0))]
        out_shape = [jax.ShapeDtypeStruct((P_out, H), f32),
                     jax.ShapeDtypeStruct((P_out, H), f32),
                     jax.ShapeDtypeStruct((1, H), f32)]

        scratch = [pltpu.VMEM((2, B2 + 8, in_dim), f32),
                   pltpu.VMEM((2, B2 + 8, H), f32),
                   pltpu.VMEM((2, B2 + 8, H), f32),
                   pltpu.VMEM((2, Bp + 8, in_dim), f32),
                   pltpu.SemaphoreType.DMA((2, 4))]

        body = functools.partial(_level_body, H, in_dim, L, C, Bp, G, s, sp,
                                 tail, only_ops, has_child)
        dh, dc, hacc = pl.pallas_call(
            body,
            grid=(G,),
            in_specs=in_specs,
            out_specs=out_specs,
            out_shape=out_shape,
            scratch_shapes=scratch,
            compiler_params=pltpu.CompilerParams(
                dimension_semantics=("arbitrary",)),
        )(*args)
        C = Lp

    x_r = _pad_rows(embed[0:1], 8)
    ss_r = _pad_rows(structure_sum[0:1], 8)
    sc_r = _pad_rows(structure_c[0:1], 8)
    mu, lv_ = pl.pallas_call(
        functools.partial(_root_body, H),
        out_shape=[jax.ShapeDtypeStruct((1, H // 2), f32),
                   jax.ShapeDtypeStruct((1, H // 2), f32)],
    )(x_r, ss_r, sc_r, dh, dc, W_iou, U_iou, b_iou2, hacc)
    return (mu, lv_)


# speedup vs baseline: 1.8253x; 1.1640x over previous
import functools
import math

import jax
import jax.numpy as jnp
from jax.experimental import pallas as pl
from jax.experimental.pallas import tpu as pltpu

_SH = 7


def _level_body(H, in_dim, L, C, Bp, G, s, sp, tail, only_ops, has_child,
                *refs):
    B2 = 2 * Bp
    it = iter(refs)
    embed = next(it)
    ss_h = next(it)
    sc_h = next(it)
    if tail:
        xt, sst, sct, xpt = next(it), next(it), next(it), next(it)
    if has_child:
        dh_in, dc_in = next(it), next(it)
    R_pair = next(it)
    W_iou, U_iou, b_iou, W_f, U_f, b_f, hacc_in = (
        next(it), next(it), next(it), next(it), next(it), next(it), next(it))
    dh_out, dc_out, hacc_out = next(it), next(it), next(it)
    x_buf, ss_buf, sc_buf, xp_buf, sems = (
        next(it), next(it), next(it), next(it), next(it))

    def copy_child(j, slot):
        def from_main():
            base = s - _SH + j * B2
            pltpu.make_async_copy(embed.at[pl.ds(base, B2 + 8), :],
                                  x_buf.at[slot], sems.at[slot, 0]).start()
            pltpu.make_async_copy(ss_h.at[pl.ds(base, B2 + 8), :],
                                  ss_buf.at[slot], sems.at[slot, 1]).start()
            pltpu.make_async_copy(sc_h.at[pl.ds(base, B2 + 8), :],
                                  sc_buf.at[slot], sems.at[slot, 2]).start()
            pltpu.make_async_copy(embed.at[pl.ds(sp - _SH + j * Bp, Bp + 8), :],
                                  xp_buf.at[slot], sems.at[slot, 3]).start()

        def from_ops():
            pltpu.make_async_copy(xt.at[:, :], x_buf.at[slot],
                                  sems.at[slot, 0]).start()
            pltpu.make_async_copy(sst.at[:, :], ss_buf.at[slot],
                                  sems.at[slot, 1]).start()
            pltpu.make_async_copy(sct.at[:, :], sc_buf.at[slot],
                                  sems.at[slot, 2]).start()
            pltpu.make_async_copy(xpt.at[:, :], xp_buf.at[slot],
                                  sems.at[slot, 3]).start()

        if only_ops:
            from_ops()
        elif tail:
            @pl.when(j < G - 1)
            def _():
                from_main()

            @pl.when(j == G - 1)
            def _():
                from_ops()
        else:
            from_main()

    def wait_child(slot):
        pltpu.make_async_copy(embed.at[pl.ds(0, B2 + 8), :],
                              x_buf.at[slot], sems.at[slot, 0]).wait()
        pltpu.make_async_copy(ss_h.at[pl.ds(0, B2 + 8), :],
                              ss_buf.at[slot], sems.at[slot, 1]).wait()
        pltpu.make_async_copy(sc_h.at[pl.ds(0, B2 + 8), :],
                              sc_buf.at[slot], sems.at[slot, 2]).wait()
        pltpu.make_async_copy(embed.at[pl.ds(0, Bp + 8), :],
                              xp_buf.at[slot], sems.at[slot, 3]).wait()

    i = pl.program_id(0)
    slot = jax.lax.rem(i, 2)

    @pl.when(i == 0)
    def _():
        copy_child(0, 0)

    @pl.when(i + 1 < G)
    def _():
        copy_child(i + 1, jax.lax.rem(i + 1, 2))

    wait_child(slot)

    rows = jax.lax.broadcasted_iota(jnp.int32, (B2, 1), 0) + i * B2
    m = rows < L

    hsum = ss_buf[slot][_SH:_SH + B2]
    cin = sc_buf[slot][_SH:_SH + B2]
    if has_child:
        m_c = rows < C
        hsum = hsum + jnp.where(m_c, dh_in[:, :], 0.0)
        cin = cin + jnp.where(m_c, dc_in[:, :], 0.0)
    x = x_buf[slot][_SH:_SH + B2]
    iou = (jnp.dot(x, W_iou[:, :], preferred_element_type=jnp.float32)
           + b_iou[:, :]
           + jnp.dot(hsum, U_iou[:, :], preferred_element_type=jnp.float32))
    i_g = iou[:, :H]
    o_g = iou[:, H:2 * H]
    u_g = iou[:, 2 * H:]
    c = jax.nn.sigmoid(i_g) * jnp.tanh(u_g) + cin
    h = jax.nn.sigmoid(o_g) * jnp.tanh(c)
    h = jnp.where(m, h, 0.0)

    xpW = jnp.dot(xp_buf[slot][_SH:_SH + Bp], W_f[:, :],
                  preferred_element_type=jnp.float32) + b_f[:, :]
    xpW2 = jnp.broadcast_to(xpW[:, None, :], (Bp, 2, H)).reshape(B2, H)
    f = jax.nn.sigmoid(xpW2 + jnp.dot(h, U_f[:, :],
                                      preferred_element_type=jnp.float32))
    fc = jnp.where(m, f * c, 0.0)
    dh_out[:, :] = jnp.dot(R_pair[:, :], h, preferred_element_type=jnp.float32)
    dc_out[:, :] = jnp.dot(R_pair[:, :], fc, preferred_element_type=jnp.float32)
    part = jnp.sum(h, axis=0, keepdims=True)

    @pl.when(i == 0)
    def _():
        hacc_out[:, :] = hacc_in[:, :] + part

    @pl.when(i > 0)
    def _():
        hacc_out[:, :] = hacc_out[:, :] + part


def _root_body(H, x, ss, sc, dh, dc, W_iou, U_iou, b_iou, hacc_in,
               mu_out, lv_out):
    hsum = ss[0:1, :] + dh[0:1, :]
    cin = sc[0:1, :] + dc[0:1, :]
    iou = (jnp.dot(x[0:1, :], W_iou[:, :], preferred_element_type=jnp.float32)
           + b_iou[:, :]
           + jnp.dot(hsum, U_iou[:, :], preferred_element_type=jnp.float32))
    i_g = iou[:, :H]
    o_g = iou[:, H:2 * H]
    u_g = iou[:, 2 * H:]
    c = jax.nn.sigmoid(i_g) * jnp.tanh(u_g) + cin
    h = jax.nn.sigmoid(o_g) * jnp.tanh(c)
    htot = hacc_in[:, :] + h
    mu_out[:, :] = htot[:, :H // 2]
    lv_out[:, :] = jnp.tanh(htot[:, H // 2:])


def _pad_rows(a, rows):
    if a.shape[0] == rows:
        return a
    return jnp.pad(a, ((0, rows - a.shape[0]), (0, 0)))


def _round_up(v, m):
    return -(-v // m) * m


def kernel(embed, edge_index, structure_sum, structure_c,
           W_iou, U_iou, b_iou, W_f, U_f, b_f):
    del edge_index
    n = embed.shape[0]
    in_dim = embed.shape[1]
    H = U_f.shape[0]
    f32 = jnp.float32

    b_iou2 = b_iou.reshape(1, 3 * H)
    b_f2 = b_f.reshape(1, H)

    def padded_slice(arr, start, rows):
        lead = max(0, -start)
        s0 = max(0, start)
        e0 = min(n, start + rows)
        return jnp.pad(arr[s0:e0], ((lead, rows - (e0 - s0) - lead), (0, 0)))

    max_d = int(math.floor(math.log2(n)))

    levels = []
    for d in range(max_d, 0, -1):
        s = 2 ** d - 1
        e = min(2 ** (d + 1) - 1, n)
        L = e - s
        Lp = (L + 1) // 2
        Bp = min(512, max(8, _round_up(Lp, 8)))
        B2 = 2 * Bp
        G = -(-Lp // Bp)
        levels.append(dict(d=d, s=s, e=e, L=L, Lp=Lp, Bp=Bp, B2=B2, G=G,
                           sp=2 ** (d - 1) - 1))
    for k, lv in enumerate(levels):
        need = levels[k + 1]["G"] * levels[k + 1]["B2"] if k + 1 < len(levels) else 8
        lv["P_out"] = max(need, lv["G"] * lv["Bp"])

    hacc = jnp.zeros((1, H), f32)
    dh = None
    dc = None
    C = 0

    hbm = pl.BlockSpec(memory_space=pltpu.MemorySpace.HBM)

    for lv in levels:
        s, L, Lp, Bp, B2, G, sp, P_out = (lv["s"], lv["L"], lv["Lp"], lv["Bp"],
                                          lv["B2"], lv["G"], lv["sp"], lv["P_out"])
        tail = (s - _SH + G * B2 + 8 > n) or (s - _SH < 0) or (sp - _SH < 0)
        only_ops = tail and G == 1
        has_child = dh is not None

        args = [embed, structure_sum, structure_c]
        in_specs = [hbm, hbm, hbm]
        if tail:
            tb = s - _SH + (G - 1) * B2
            args += [padded_slice(embed, tb, B2 + 8),
                     padded_slice(structure_sum, tb, B2 + 8),
                     padded_slice(structure_c, tb, B2 + 8),
                     padded_slice(embed, sp - _SH + (G - 1) * Bp, Bp + 8)]
            in_specs += [hbm, hbm, hbm, hbm]
        if has_child:
            args += [dh, dc]
            in_specs += [pl.BlockSpec((B2, H), lambda i: (i, 0)),
                         pl.BlockSpec((B2, H), lambda i: (i, 0))]
        R_pair = jnp.equal(jnp.arange(B2)[None, :] // 2,
                           jnp.arange(Bp)[:, None]).astype(f32)
        args += [R_pair, W_iou, U_iou, b_iou2, W_f, U_f, b_f2, hacc]
        in_specs += [pl.BlockSpec((Bp, B2), lambda i: (0, 0))]
        in_specs += [pl.BlockSpec((in_dim, 3 * H), lambda i: (0, 0)),
                     pl.BlockSpec((H, 3 * H), lambda i: (0, 0)),
                     pl.BlockSpec((1, 3 * H), lambda i: (0, 0)),
                     pl.BlockSpec((in_dim, H), lambda i: (0, 0)),
                     pl.BlockSpec((H, H), lambda i: (0, 0)),
                     pl.BlockSpec((1, H), lambda i: (0, 0)),
                     pl.BlockSpec((1, H), lambda i: (0, 0))]

        out_specs = [pl.BlockSpec((Bp, H), lambda i: (i, 0)),
                     pl.BlockSpec((Bp, H), lambda i: (i, 0)),
                     pl.BlockSpec((1, H), lambda i: (0, 0))]
        out_shape = [jax.ShapeDtypeStruct((P_out, H), f32),
                     jax.ShapeDtypeStruct((P_out, H), f32),
                     jax.ShapeDtypeStruct((1, H), f32)]

        scratch = [pltpu.VMEM((2, B2 + 8, in_dim), f32),
                   pltpu.VMEM((2, B2 + 8, H), f32),
                   pltpu.VMEM((2, B2 + 8, H), f32),
                   pltpu.VMEM((2, Bp + 8, in_dim), f32),
                   pltpu.SemaphoreType.DMA((2, 4))]

        body = functools.partial(_level_body, H, in_dim, L, C, Bp, G, s, sp,
                                 tail, only_ops, has_child)
        dh, dc, hacc = pl.pallas_call(
            body,
            grid=(G,),
            in_specs=in_specs,
            out_specs=out_specs,
            out_shape=out_shape,
            scratch_shapes=scratch,
            compiler_params=pltpu.CompilerParams(
                dimension_semantics=("arbitrary",)),
        )(*args)
        C = Lp

    x_r = _pad_rows(embed[0:1], 8)
    ss_r = _pad_rows(structure_sum[0:1], 8)
    sc_r = _pad_rows(structure_c[0:1], 8)
    mu, lv_ = pl.pallas_call(
        functools.partial(_root_body, H),
        out_shape=[jax.ShapeDtypeStruct((1, H // 2), f32),
                   jax.ShapeDtypeStruct((1, H // 2), f32)],
    )(x_r, ss_r, sc_r, dh, dc, W_iou, U_iou, b_iou2, hacc)
    return (mu, lv_)


# speedup vs baseline: 2.2140x; 1.2129x over previous
import functools
import math

import jax
import jax.numpy as jnp
from jax.experimental import pallas as pl
from jax.experimental.pallas import tpu as pltpu

_SH = 7


def _sel(table, idx):
    v = jnp.int32(table[0])
    for k in range(1, len(table)):
        v = jnp.where(idx >= k, jnp.int32(table[k]), v)
    return v


def _big_body(H, in_dim, Bp, tbl, tail_step, n_steps,
              embed, ss_h, sc_h, xt, sst, sct, xpt,
              R_pair, W_iou, U_iou, b_iou, W_f, U_f, b_f,
              dh_fin, dc_fin, hacc_out,
              x_buf, ss_buf, sc_buf, xp_buf, dh_buf, dc_buf, sems):
    B2 = 2 * Bp

    def params(step):
        lv = jnp.int32(0)
        for st in tbl["start"][1:]:
            lv = lv + jnp.where(step >= st, 1, 0).astype(jnp.int32)
        j = step - _sel(tbl["start"], lv)
        return lv, j

    def copy_in(step, slot):
        lv, j = params(step)
        cb8 = _sel(tbl["s8"], lv) + j * (B2 // 8)
        pb8 = _sel(tbl["sp8"], lv) + j * (Bp // 8)
        cb = pl.multiple_of(cb8 * 8, 8)
        pb = pl.multiple_of(pb8 * 8, 8)

        @pl.when(step != tail_step)
        def _():
            pltpu.make_async_copy(embed.at[pl.ds(cb, B2 + 8), :],
                                  x_buf.at[slot], sems.at[slot, 0]).start()
            pltpu.make_async_copy(ss_h.at[pl.ds(cb, B2 + 8), :],
                                  ss_buf.at[slot], sems.at[slot, 1]).start()
            pltpu.make_async_copy(sc_h.at[pl.ds(cb, B2 + 8), :],
                                  sc_buf.at[slot], sems.at[slot, 2]).start()
            pltpu.make_async_copy(embed.at[pl.ds(pb, Bp + 8), :],
                                  xp_buf.at[slot], sems.at[slot, 3]).start()

        @pl.when(step == tail_step)
        def _():
            pltpu.make_async_copy(xt.at[:, :], x_buf.at[slot],
                                  sems.at[slot, 0]).start()
            pltpu.make_async_copy(sst.at[:, :], ss_buf.at[slot],
                                  sems.at[slot, 1]).start()
            pltpu.make_async_copy(sct.at[:, :], sc_buf.at[slot],
                                  sems.at[slot, 2]).start()
            pltpu.make_async_copy(xpt.at[:, :], xp_buf.at[slot],
                                  sems.at[slot, 3]).start()

    def wait_in(slot):
        pltpu.make_async_copy(embed.at[pl.ds(0, B2 + 8), :],
                              x_buf.at[slot], sems.at[slot, 0]).wait()
        pltpu.make_async_copy(ss_h.at[pl.ds(0, B2 + 8), :],
                              ss_buf.at[slot], sems.at[slot, 1]).wait()
        pltpu.make_async_copy(sc_h.at[pl.ds(0, B2 + 8), :],
                              sc_buf.at[slot], sems.at[slot, 2]).wait()
        pltpu.make_async_copy(embed.at[pl.ds(0, Bp + 8), :],
                              xp_buf.at[slot], sems.at[slot, 3]).wait()

    i = pl.program_id(0)
    slot = jax.lax.rem(i, 2)

    @pl.when(i == 0)
    def _():
        copy_in(0, 0)

    @pl.when(i + 1 < n_steps)
    def _():
        copy_in(i + 1, jax.lax.rem(i + 1, 2))

    wait_in(slot)

    lv, j = params(i)
    p = jax.lax.rem(lv, 2)
    L_rem = _sel(tbl["L"], lv) - j * B2
    C_rem = _sel(tbl["C"], lv) - j * B2
    rd8 = jnp.minimum(j * (B2 // 8), _sel(tbl["cap8"], lv))
    rd = pl.multiple_of(rd8 * 8, 8)
    wr = pl.multiple_of(j * Bp, Bp)

    rows = jax.lax.broadcasted_iota(jnp.int32, (B2, 1), 0)
    m = rows < L_rem
    m_c = rows < C_rem

    hsum = ss_buf[slot][_SH:_SH + B2]
    cin = sc_buf[slot][_SH:_SH + B2]
    hsum = hsum + jnp.where(m_c, dh_buf[1 - p, pl.ds(rd, B2), :], 0.0)
    cin = cin + jnp.where(m_c, dc_buf[1 - p, pl.ds(rd, B2), :], 0.0)
    x = x_buf[slot][_SH:_SH + B2]
    iou = (jnp.dot(x, W_iou[:, :], preferred_element_type=jnp.float32)
           + b_iou[:, :]
           + jnp.dot(hsum, U_iou[:, :], preferred_element_type=jnp.float32))
    c = jax.nn.sigmoid(iou[:, :H]) * jnp.tanh(iou[:, 2 * H:]) + cin
    h = jax.nn.sigmoid(iou[:, H:2 * H]) * jnp.tanh(c)
    h = jnp.where(m, h, 0.0)

    xpW = jnp.dot(xp_buf[slot][_SH:_SH + Bp], W_f[:, :],
                  preferred_element_type=jnp.float32) + b_f[:, :]
    xpW2 = jnp.broadcast_to(xpW[:, None, :], (Bp, 2, H)).reshape(B2, H)
    f = jax.nn.sigmoid(xpW2 + jnp.dot(h, U_f[:, :],
                                      preferred_element_type=jnp.float32))
    fc = jnp.where(m, f * c, 0.0)
    dh_t = jnp.dot(R_pair[:, :], h, preferred_element_type=jnp.float32)
    dc_t = jnp.dot(R_pair[:, :], fc, preferred_element_type=jnp.float32)
    dh_buf[p, pl.ds(wr, Bp), :] = dh_t
    dc_buf[p, pl.ds(wr, Bp), :] = dc_t
    part = jnp.sum(h, axis=0, keepdims=True)

    @pl.when(i == 0)
    def _():
        hacc_out[:, :] = part

    @pl.when(i > 0)
    def _():
        hacc_out[:, :] = hacc_out[:, :] + part

    @pl.when(i == n_steps - 1)
    def _():
        dh_fin[:, :] = dh_t
        dc_fin[:, :] = dc_t


def _top_body(H, top_d, x_all, ss_all, sc_all, dh_in, dc_in,
              W_iou, U_iou, b_iou, W_f, U_f, b_f, hacc_in, mu_out, lv_out):
    Wi = W_iou[:, :]
    Ui = U_iou[:, :]
    bi = b_iou[:, :]
    Wf = W_f[:, :]
    Uf = U_f[:, :]
    bf = b_f[:, :]
    xv = x_all[:, :]
    ssv = ss_all[:, :]
    scv = sc_all[:, :]
    hacc = hacc_in[:, :]
    dh = dh_in[:, :]
    dc = dc_in[:, :]

    for d in range(top_d, -1, -1):
        s = 2 ** d - 1
        L = 2 ** d
        x_l = xv[s:s + L]
        hs = ssv[s:s + L] + dh[:L]
        cn = scv[s:s + L] + dc[:L]
        iou = (jnp.dot(x_l, Wi, preferred_element_type=jnp.float32) + bi
               + jnp.dot(hs, Ui, preferred_element_type=jnp.float32))
        c = jax.nn.sigmoid(iou[:, :H]) * jnp.tanh(iou[:, 2 * H:]) + cn
        h = jax.nn.sigmoid(iou[:, H:2 * H]) * jnp.tanh(c)
        hacc = hacc + jnp.sum(h, axis=0, keepdims=True)
        if d == 0:
            break
        Lp = L // 2
        sp = 2 ** (d - 1) - 1
        xpW = jnp.dot(xv[sp:sp + Lp], Wf,
                      preferred_element_type=jnp.float32) + bf
        xpW2 = jnp.broadcast_to(xpW[:, None, :], (Lp, 2, H)).reshape(L, H)
        f = jax.nn.sigmoid(xpW2 + jnp.dot(h, Uf,
                                          preferred_element_type=jnp.float32))
        dh = h.reshape(Lp, 2, H).sum(axis=1)
        dc = (f * c).reshape(Lp, 2, H).sum(axis=1)

    mu_out[:, :] = hacc[:, :H // 2]
    lv_out[:, :] = jnp.tanh(hacc[:, H // 2:])


def _pad_rows(a, rows):
    if a.shape[0] == rows:
        return a
    return jnp.pad(a, ((0, rows - a.shape[0]), (0, 0)))


def kernel(embed, edge_index, structure_sum, structure_c,
           W_iou, U_iou, b_iou, W_f, U_f, b_f):
    del edge_index
    n = embed.shape[0]
    in_dim = embed.shape[1]
    H = U_f.shape[0]
    f32 = jnp.float32

    b_iou2 = b_iou.reshape(1, 3 * H)
    b_f2 = b_f.reshape(1, H)

    def padded_slice(arr, start, rows):
        lead = max(0, -start)
        s0 = max(0, start)
        e0 = min(n, start + rows)
        return jnp.pad(arr[s0:e0], ((lead, rows - (e0 - s0) - lead), (0, 0)))

    max_d = int(math.floor(math.log2(n)))
    top_d = 9
    Bp = 512
    B2 = 2 * Bp

    s_l, sp_l, L_l, C_l, G_l = [], [], [], [], []
    prev_Lp = 0
    for d in range(max_d, top_d, -1):
        s = 2 ** d - 1
        e = min(2 ** (d + 1) - 1, n)
        L = e - s
        Lp = (L + 1) // 2
        s_l.append(s)
        sp_l.append(2 ** (d - 1) - 1)
        L_l.append(L)
        C_l.append(prev_Lp)
        G_l.append(-(-Lp // Bp))
        prev_Lp = Lp
    nlv = len(s_l)
    start = [0]
    for g in G_l:
        start.append(start[-1] + g)
    n_steps = start[-1]
    start = start[:-1]
    cap8 = [0] + [max(0, G_l[k - 1] * Bp - B2) // 8 for k in range(1, nlv)]
    buf_rows = max(g * Bp for g in G_l)
    tbl = {
        "start": start,
        "s8": [(s - _SH) // 8 for s in s_l],
        "sp8": [(sp - _SH) // 8 for sp in sp_l],
        "L": L_l,
        "C": C_l,
        "cap8": cap8,
    }
    tail_step = G_l[0] - 1
    tb = s_l[0] - _SH + tail_step * B2
    x_t = padded_slice(embed, tb, B2 + 8)
    ss_t = padded_slice(structure_sum, tb, B2 + 8)
    sc_t = padded_slice(structure_c, tb, B2 + 8)
    xp_t = padded_slice(embed, sp_l[0] - _SH + tail_step * Bp, Bp + 8)

    R_pair = jnp.equal(jnp.arange(B2)[None, :] // 2,
                       jnp.arange(Bp)[:, None]).astype(f32)

    hbm = pl.BlockSpec(memory_space=pltpu.MemorySpace.HBM)

    def full(shape):
        return pl.BlockSpec(shape, lambda i: (0,) * len(shape))

    dh, dc, hacc = pl.pallas_call(
        functools.partial(_big_body, H, in_dim, Bp, tbl, tail_step, n_steps),
        grid=(n_steps,),
        in_specs=[hbm, hbm, hbm, hbm, hbm, hbm, hbm,
                  full((Bp, B2)),
                  full((in_dim, 3 * H)), full((H, 3 * H)), full((1, 3 * H)),
                  full((in_dim, H)), full((H, H)), full((1, H))],
        out_specs=[full((Bp, H)), full((Bp, H)), full((1, H))],
        out_shape=[jax.ShapeDtypeStruct((Bp, H), f32),
                   jax.ShapeDtypeStruct((Bp, H), f32),
                   jax.ShapeDtypeStruct((1, H), f32)],
        scratch_shapes=[pltpu.VMEM((2, B2 + 8, in_dim), f32),
                        pltpu.VMEM((2, B2 + 8, H), f32),
                        pltpu.VMEM((2, B2 + 8, H), f32),
                        pltpu.VMEM((2, Bp + 8, in_dim), f32),
                        pltpu.VMEM((2, buf_rows, H), f32),
                        pltpu.VMEM((2, buf_rows, H), f32),
                        pltpu.SemaphoreType.DMA((2, 4))],
        compiler_params=pltpu.CompilerParams(
            dimension_semantics=("arbitrary",)),
    )(embed, structure_sum, structure_c, x_t, ss_t, sc_t, xp_t,
      R_pair, W_iou, U_iou, b_iou2, W_f, U_f, b_f2)

    n_top = 2 ** (top_d + 1) - 1
    x_a = _pad_rows(embed[0:n_top], n_top + 1)
    ss_a = _pad_rows(structure_sum[0:n_top], n_top + 1)
    sc_a = _pad_rows(structure_c[0:n_top], n_top + 1)
    mu, lv_ = pl.pallas_call(
        functools.partial(_top_body, H, top_d),
        out_shape=[jax.ShapeDtypeStruct((1, H // 2), f32),
                   jax.ShapeDtypeStruct((1, H // 2), f32)],
    )(x_a, ss_a, sc_a, dh, dc, W_iou, U_iou, b_iou2, W_f, U_f, b_f2, hacc)
    return (mu, lv_)
